# baseline (speedup 1.0000x reference)
import functools

import jax
import jax.numpy as jnp
from jax import lax
from jax.experimental import pallas as pl
from jax.experimental.pallas import tpu as pltpu
from jax.experimental.pallas import tpu_sc as plsc

MIN_T2 = 0.02 * 0.02
BIG2 = 9999.0 * 9999.0

NS_PAD = 7168
GROUPS = 64
C_SUB = 8
S_LANES = 256
UNROLL = 14
C_PER_PROG = GROUPS * C_SUB
N_CTILES = 8192 // C_PER_PROG


def _dist_kernel(cloth_ref, srep_ref, m_ref, idx_ref, cb_ref):
    bid = pl.program_id(0)
    c3 = cloth_ref[0]
    for g in range(GROUPS):
        for c in range(3):
            cb_ref[pl.ds((g * 3 + c) * C_SUB, C_SUB), :] = jnp.broadcast_to(
                c3[g * C_SUB:(g + 1) * C_SUB, c:c + 1], (C_SUB, S_LANES))
    lane_rep = lax.broadcasted_iota(jnp.int32, (C_SUB, S_LANES), 1)

    def body(k, carry):
        ms, iss = carry
        new_ms = list(ms)
        new_is = list(iss)
        sxs = []
        sys_ = []
        szs = []
        ics = []
        for u in range(UNROLL):
            off = (k * UNROLL + u) * S_LANES
            sxs.append(srep_ref[0, 0:C_SUB, pl.ds(off, S_LANES)])
            sys_.append(srep_ref[0, C_SUB:2 * C_SUB, pl.ds(off, S_LANES)])
            szs.append(srep_ref[0, 2 * C_SUB:3 * C_SUB, pl.ds(off, S_LANES)])
            ics.append(lane_rep + off)
        for g in range(GROUPS):
            cxg = cb_ref[pl.ds((g * 3 + 0) * C_SUB, C_SUB), :]
            cyg = cb_ref[pl.ds((g * 3 + 1) * C_SUB, C_SUB), :]
            czg = cb_ref[pl.ds((g * 3 + 2) * C_SUB, C_SUB), :]
            mg = new_ms[g]
            ig = new_is[g]
            for u in range(UNROLL):
                dx = cxg - sxs[u]
                dy = cyg - sys_[u]
                dz = czg - szs[u]
                d2 = dx * dx + dy * dy + dz * dz
                d2 = jnp.where(d2 < MIN_T2, BIG2, d2)
                upd = d2 < mg
                ig = jnp.where(upd, ics[u], ig)
                mg = jnp.where(upd, d2, mg)
            new_ms[g] = mg
            new_is[g] = ig
        return tuple(new_ms), tuple(new_is)

    m0 = tuple(jnp.full((C_SUB, S_LANES), jnp.inf, jnp.float32)
               for _ in range(GROUPS))
    i0 = tuple(jnp.zeros((C_SUB, S_LANES), jnp.int32)
               for _ in range(GROUPS))
    ms, iss = lax.fori_loop(0, NS_PAD // (S_LANES * UNROLL), body, (m0, i0))

    big_i = jnp.int32(2 ** 30)
    for g in range(GROUPS):
        m = jnp.min(ms[g], axis=1, keepdims=True)
        isel = jnp.min(jnp.where(ms[g] == m, iss[g], big_i), axis=1,
                       keepdims=True)
        m_ref[0, 0, pl.ds(g * C_SUB, C_SUB)] = m
        idx_ref[0, 0, pl.ds(g * C_SUB, C_SUB)] = isel + bid * NS_PAD


def _nearest_v3(cloth, srep):
    B = cloth.shape[0]
    grid = (B, N_CTILES)
    out_shape = [
        jax.ShapeDtypeStruct((B, N_CTILES, C_PER_PROG, 1), jnp.float32),
        jax.ShapeDtypeStruct((B, N_CTILES, C_PER_PROG, 1), jnp.int32),
    ]
    m, idx = pl.pallas_call(
        _dist_kernel,
        grid=grid,
        in_specs=[
            pl.BlockSpec((1, C_PER_PROG, 3), lambda b, c: (b, c, 0)),
            pl.BlockSpec((1, 3 * C_SUB, NS_PAD), lambda b, c: (b, 0, 0)),
        ],
        out_specs=[
            pl.BlockSpec((1, 1, C_PER_PROG, 1), lambda b, c: (b, c, 0, 0)),
            pl.BlockSpec((1, 1, C_PER_PROG, 1), lambda b, c: (b, c, 0, 0)),
        ],
        out_shape=out_shape,
        scratch_shapes=[pltpu.VMEM((3 * C_PER_PROG, S_LANES), jnp.float32)],
        compiler_params=pltpu.CompilerParams(
            dimension_semantics=("parallel", "parallel"),
        ),
    )(cloth, srep)
    return m.reshape(B, -1), idx.reshape(B, -1)


N_IROWS = 8192 // 128


def _sc_loss_kernel(m_hbm, idx_hbm, sdf_hbm, lab_hbm, cvec_hbm, dt_hbm, st_hbm,
                    out_hbm, idx_v, gath_v, m_v, sdf_v, sc_v, sem):
    NC_SC = 8192
    cid = lax.axis_index("c")
    sid = lax.axis_index("s")
    wid = cid * 16 + sid

    @pl.when(wid < 8)
    def _():
        pltpu.sync_copy(idx_hbm.at[pl.ds(wid * N_IROWS, N_IROWS)], idx_v)
        pltpu.sync_copy(m_hbm.at[pl.ds(wid * NC_SC, NC_SC)], m_v)
        pltpu.sync_copy(sdf_hbm.at[pl.ds(wid * NC_SC, NC_SC)], sdf_v)
        pltpu.sync_copy(cvec_hbm, sc_v.at[0])
        pltpu.sync_copy(dt_hbm, sc_v.at[1])
        pltpu.sync_copy(st_hbm, sc_v.at[2])

        copies = [
            pltpu.async_copy(lab_hbm.at[idx_v.at[j]], gath_v.at[j], sem)
            for j in range(N_IROWS)
        ]
        for c in copies:
            c.wait()

        cvec = sc_v[0]
        dt = sc_v[1]
        st = sc_v[2]
        dt2 = dt * dt

        def body(j, carry):
            acc, cnt = carry
            for k in range(8):
                lab = gath_v[j, pl.ds(k * 16, 16)]
                sl = pl.ds(j * 128 + k * 16, 16)
                mf = jnp.where(lab == cvec, 1.0, 0.0).astype(jnp.float32)
                d2 = m_v[sl]
                s = sdf_v[sl]
                nf = jnp.where(d2 < dt2, 1.0, 0.0).astype(jnp.float32)
                lp = jnp.abs(s) * mf
                ln = jnp.abs(s - st) * (1.0 - mf)
                acc = acc + (lp + ln) * nf
                cnt = cnt + mf
            return acc, cnt

        z = jnp.zeros((16,), jnp.float32)
        acc, cnt = lax.fori_loop(0, N_IROWS, body, (z, z))
        sc_v[4] = acc
        sc_v[5] = cnt
        pltpu.sync_copy(sc_v.at[pl.ds(4, 2)], out_hbm.at[wid])


def _sc_loss(m, idx, sdf, lab, cvec, dtv, stv):
    B = sdf.shape[0]
    mesh = plsc.VectorSubcoreMesh(core_axis_name="c", subcore_axis_name="s")
    fn = functools.partial(
        pl.kernel,
        mesh=mesh,
        out_type=jax.ShapeDtypeStruct((B, 2, 16), jnp.float32),
        scratch_types=[
            pltpu.VMEM((N_IROWS, 128), jnp.int32),
            pltpu.VMEM((N_IROWS, 128), jnp.float32),
            pltpu.VMEM((8192,), jnp.float32),
            pltpu.VMEM((8192,), jnp.float32),
            pltpu.VMEM((6, 16), jnp.float32),
            pltpu.SemaphoreType.DMA,
        ],
    )(_sc_loss_kernel)
    out = fn(m.reshape(-1), idx.reshape(B * N_IROWS, 128), sdf.reshape(-1),
             lab.reshape(-1), cvec, dtv, stv)
    total = out[:, 0, :].sum(axis=1)
    n_in = out[:, 1, :].sum(axis=1)
    return total * (1.0 / 8192.0) * (n_in > 0.0).astype(jnp.float32)


def kernel(sdf, cloth_meshes_unposed, smpl_cloth_idx, smpl_cloth_valid,
           cloth_idx, sdf_thresh, dist_thresh, v_template):
    B, Nc, _ = cloth_meshes_unposed.shape
    Ns = v_template.shape[1]
    pad = NS_PAD - Ns

    FAR = jnp.float32(1e18)
    masked = jnp.where((smpl_cloth_valid > 0)[:, :, None], v_template, FAR)
    masked = jnp.pad(masked, ((0, 0), (0, pad), (0, 0)),
                     constant_values=1e18)
    smplt = jnp.swapaxes(masked, 1, 2)
    srep = jnp.broadcast_to(smplt[:, :, None, :],
                            (B, 3, C_SUB, NS_PAD)).reshape(B, 3 * C_SUB,
                                                           NS_PAD)

    m, idx = _nearest_v3(cloth_meshes_unposed, srep)

    lab = jnp.pad(smpl_cloth_idx, ((0, 0), (0, pad))).astype(jnp.float32)
    cvec = jnp.broadcast_to(cloth_idx[0].astype(jnp.float32), (16,))
    dtv = jnp.broadcast_to(dist_thresh.astype(jnp.float32), (16,))
    stv = jnp.broadcast_to(sdf_thresh.astype(jnp.float32), (16,))

    return _sc_loss(m, idx, sdf, lab, cvec, dtv, stv)


# speedup vs baseline: 1.0324x; 1.0324x over previous
import functools

import jax
import jax.numpy as jnp
from jax import lax
from jax.experimental import pallas as pl
from jax.experimental.pallas import tpu as pltpu
from jax.experimental.pallas import tpu_sc as plsc

MIN_T2 = 0.02 * 0.02
BIG2 = 9999.0 * 9999.0

NS_PAD = 7168
GROUPS = 64
C_SUB = 8
S_LANES = 128
UNROLL = 28
C_PER_PROG = GROUPS * C_SUB
N_CTILES = 8192 // C_PER_PROG


def _dist_kernel(cloth_ref, srep_ref, m_ref, idx_ref, cb_ref):
    bid = pl.program_id(0)
    c3 = cloth_ref[0]
    for g in range(GROUPS):
        for c in range(3):
            cb_ref[pl.ds((g * 3 + c) * C_SUB, C_SUB), :] = jnp.broadcast_to(
                c3[g * C_SUB:(g + 1) * C_SUB, c:c + 1], (C_SUB, S_LANES))
    lane_rep = lax.broadcasted_iota(jnp.int32, (C_SUB, S_LANES), 1)

    def body(k, carry):
        ms, iss = carry
        new_ms = list(ms)
        new_is = list(iss)
        sxs = []
        sys_ = []
        szs = []
        ics = []
        for u in range(UNROLL):
            off = (k * UNROLL + u) * S_LANES
            sxs.append(srep_ref[0, 0:C_SUB, pl.ds(off, S_LANES)])
            sys_.append(srep_ref[0, C_SUB:2 * C_SUB, pl.ds(off, S_LANES)])
            szs.append(srep_ref[0, 2 * C_SUB:3 * C_SUB, pl.ds(off, S_LANES)])
            ics.append(lane_rep + off)
        for g in range(GROUPS):
            cxg = cb_ref[pl.ds((g * 3 + 0) * C_SUB, C_SUB), :]
            cyg = cb_ref[pl.ds((g * 3 + 1) * C_SUB, C_SUB), :]
            czg = cb_ref[pl.ds((g * 3 + 2) * C_SUB, C_SUB), :]
            mg = new_ms[g]
            ig = new_is[g]
            for u in range(UNROLL):
                dx = cxg - sxs[u]
                dy = cyg - sys_[u]
                dz = czg - szs[u]
                d2 = dx * dx + dy * dy + dz * dz
                d2 = jnp.where(d2 < MIN_T2, BIG2, d2)
                upd = d2 < mg
                ig = jnp.where(upd, ics[u], ig)
                mg = jnp.where(upd, d2, mg)
            new_ms[g] = mg
            new_is[g] = ig
        return tuple(new_ms), tuple(new_is)

    m0 = tuple(jnp.full((C_SUB, S_LANES), jnp.inf, jnp.float32)
               for _ in range(GROUPS))
    i0 = tuple(jnp.zeros((C_SUB, S_LANES), jnp.int32)
               for _ in range(GROUPS))
    ms, iss = lax.fori_loop(0, NS_PAD // (S_LANES * UNROLL), body, (m0, i0))

    big_i = jnp.int32(2 ** 30)
    for g in range(GROUPS):
        m = jnp.min(ms[g], axis=1, keepdims=True)
        isel = jnp.min(jnp.where(ms[g] == m, iss[g], big_i), axis=1,
                       keepdims=True)
        m_ref[0, 0, pl.ds(g * C_SUB, C_SUB)] = m
        idx_ref[0, 0, pl.ds(g * C_SUB, C_SUB)] = isel + bid * NS_PAD


def _nearest_v3(cloth, srep):
    B = cloth.shape[0]
    grid = (B, N_CTILES)
    out_shape = [
        jax.ShapeDtypeStruct((B, N_CTILES, C_PER_PROG, 1), jnp.float32),
        jax.ShapeDtypeStruct((B, N_CTILES, C_PER_PROG, 1), jnp.int32),
    ]
    m, idx = pl.pallas_call(
        _dist_kernel,
        grid=grid,
        in_specs=[
            pl.BlockSpec((1, C_PER_PROG, 3), lambda b, c: (b, c, 0)),
            pl.BlockSpec((1, 3 * C_SUB, NS_PAD), lambda b, c: (b, 0, 0)),
        ],
        out_specs=[
            pl.BlockSpec((1, 1, C_PER_PROG, 1), lambda b, c: (b, c, 0, 0)),
            pl.BlockSpec((1, 1, C_PER_PROG, 1), lambda b, c: (b, c, 0, 0)),
        ],
        out_shape=out_shape,
        scratch_shapes=[pltpu.VMEM((3 * C_PER_PROG, S_LANES), jnp.float32)],
        compiler_params=pltpu.CompilerParams(
            dimension_semantics=("parallel", "parallel"),
        ),
    )(cloth, srep)
    return m.reshape(B, -1), idx.reshape(B, -1)


N_IROWS = 8192 // 128


N_TILES_SC = 32
ROWS_PER_TILE = N_IROWS * 8 // N_TILES_SC
ELEMS_PER_TILE = ROWS_PER_TILE * 128


def _sc_loss_kernel(m_hbm, idx_hbm, sdf_hbm, lab_hbm, cvec_hbm, dt_hbm, st_hbm,
                    out_hbm, idx_v, gath_v, m_v, sdf_v, sc_v, sem):
    cid = lax.axis_index("c")
    sid = lax.axis_index("s")
    wid = cid * 16 + sid

    pltpu.sync_copy(idx_hbm.at[pl.ds(wid * ROWS_PER_TILE, ROWS_PER_TILE)],
                    idx_v)
    pltpu.sync_copy(m_hbm.at[pl.ds(wid * ELEMS_PER_TILE, ELEMS_PER_TILE)], m_v)
    pltpu.sync_copy(sdf_hbm.at[pl.ds(wid * ELEMS_PER_TILE, ELEMS_PER_TILE)],
                    sdf_v)
    pltpu.sync_copy(cvec_hbm, sc_v.at[0])
    pltpu.sync_copy(dt_hbm, sc_v.at[1])
    pltpu.sync_copy(st_hbm, sc_v.at[2])

    copies = [
        pltpu.async_copy(lab_hbm.at[idx_v.at[j]], gath_v.at[j], sem)
        for j in range(ROWS_PER_TILE)
    ]
    for c in copies:
        c.wait()

    cvec = sc_v[0]
    dt = sc_v[1]
    st = sc_v[2]
    dt2 = dt * dt

    def body(j, carry):
        acc, cnt = carry
        for k in range(8):
            lab = gath_v[j, pl.ds(k * 16, 16)]
            sl = pl.ds(j * 128 + k * 16, 16)
            mf = jnp.where(lab == cvec, 1.0, 0.0).astype(jnp.float32)
            d2 = m_v[sl]
            s = sdf_v[sl]
            nf = jnp.where(d2 < dt2, 1.0, 0.0).astype(jnp.float32)
            lp = jnp.abs(s) * mf
            ln = jnp.abs(s - st) * (1.0 - mf)
            acc = acc + (lp + ln) * nf
            cnt = cnt + mf
        return acc, cnt

    z = jnp.zeros((16,), jnp.float32)
    acc, cnt = lax.fori_loop(0, ROWS_PER_TILE, body, (z, z))
    sc_v[4] = acc
    sc_v[5] = cnt
    pltpu.sync_copy(sc_v.at[pl.ds(4, 2)], out_hbm.at[wid])


def _sc_loss(m, idx, sdf, lab, cvec, dtv, stv):
    B = sdf.shape[0]
    mesh = plsc.VectorSubcoreMesh(core_axis_name="c", subcore_axis_name="s")
    fn = functools.partial(
        pl.kernel,
        mesh=mesh,
        out_type=jax.ShapeDtypeStruct((N_TILES_SC, 2, 16), jnp.float32),
        scratch_types=[
            pltpu.VMEM((ROWS_PER_TILE, 128), jnp.int32),
            pltpu.VMEM((ROWS_PER_TILE, 128), jnp.float32),
            pltpu.VMEM((ELEMS_PER_TILE,), jnp.float32),
            pltpu.VMEM((ELEMS_PER_TILE,), jnp.float32),
            pltpu.VMEM((6, 16), jnp.float32),
            pltpu.SemaphoreType.DMA,
        ],
    )(_sc_loss_kernel)
    out = fn(m.reshape(-1), idx.reshape(B * N_IROWS, 128), sdf.reshape(-1),
             lab.reshape(-1), cvec, dtv, stv)
    out = out.reshape(B, N_TILES_SC // B, 2, 16)
    total = out[:, :, 0, :].sum(axis=(1, 2))
    n_in = out[:, :, 1, :].sum(axis=(1, 2))
    return total * (1.0 / 8192.0) * (n_in > 0.0).astype(jnp.float32)


def kernel(sdf, cloth_meshes_unposed, smpl_cloth_idx, smpl_cloth_valid,
           cloth_idx, sdf_thresh, dist_thresh, v_template):
    B, Nc, _ = cloth_meshes_unposed.shape
    Ns = v_template.shape[1]
    pad = NS_PAD - Ns

    FAR = jnp.float32(1e18)
    masked = jnp.where((smpl_cloth_valid > 0)[:, :, None], v_template, FAR)
    masked = jnp.pad(masked, ((0, 0), (0, pad), (0, 0)),
                     constant_values=1e18)
    smplt = jnp.swapaxes(masked, 1, 2)
    srep = jnp.broadcast_to(smplt[:, :, None, :],
                            (B, 3, C_SUB, NS_PAD)).reshape(B, 3 * C_SUB,
                                                           NS_PAD)

    m, idx = _nearest_v3(cloth_meshes_unposed, srep)

    lab = jnp.pad(smpl_cloth_idx, ((0, 0), (0, pad))).astype(jnp.float32)
    cvec = jnp.broadcast_to(cloth_idx[0].astype(jnp.float32), (16,))
    dtv = jnp.broadcast_to(dist_thresh.astype(jnp.float32), (16,))
    stv = jnp.broadcast_to(sdf_thresh.astype(jnp.float32), (16,))

    return _sc_loss(m, idx, sdf, lab, cvec, dtv, stv)


# speedup vs baseline: 1.0330x; 1.0006x over previous
import functools

import jax
import jax.numpy as jnp
from jax import lax
from jax.experimental import pallas as pl
from jax.experimental.pallas import tpu as pltpu
from jax.experimental.pallas import tpu_sc as plsc

MIN_T2 = 0.02 * 0.02
BIG2 = 9999.0 * 9999.0

NS_PAD = 7168
GROUPS = 64
C_SUB = 8
S_LANES = 128
UNROLL = 28
C_PER_PROG = GROUPS * C_SUB
N_CTILES = 8192 // C_PER_PROG


def _dist_kernel(cloth_ref, srep_ref, m_ref, idx_ref, cb_ref):
    bid = pl.program_id(0)
    c3 = cloth_ref[0]
    for g in range(GROUPS):
        for c in range(3):
            cb_ref[pl.ds((g * 3 + c) * C_SUB, C_SUB), :] = jnp.broadcast_to(
                c3[g * C_SUB:(g + 1) * C_SUB, c:c + 1], (C_SUB, S_LANES))
    lane_rep = lax.broadcasted_iota(jnp.int32, (C_SUB, S_LANES), 1)

    def body(k, carry):
        ms, iss = carry
        new_ms = list(ms)
        new_is = list(iss)
        sxs = []
        sys_ = []
        szs = []
        ics = []
        for u in range(UNROLL):
            off = (k * UNROLL + u) * S_LANES
            sxs.append(srep_ref[0, 0:C_SUB, pl.ds(off, S_LANES)])
            sys_.append(srep_ref[0, C_SUB:2 * C_SUB, pl.ds(off, S_LANES)])
            szs.append(srep_ref[0, 2 * C_SUB:3 * C_SUB, pl.ds(off, S_LANES)])
            ics.append(lane_rep + off)
        for g in range(GROUPS):
            cxg = cb_ref[pl.ds((g * 3 + 0) * C_SUB, C_SUB), :]
            cyg = cb_ref[pl.ds((g * 3 + 1) * C_SUB, C_SUB), :]
            czg = cb_ref[pl.ds((g * 3 + 2) * C_SUB, C_SUB), :]
            mg = new_ms[g]
            ig = new_is[g]
            for u in range(UNROLL):
                dx = cxg - sxs[u]
                dy = cyg - sys_[u]
                dz = czg - szs[u]
                d2 = dx * dx + dy * dy + dz * dz
                d2 = jnp.where(d2 < MIN_T2, BIG2, d2)
                upd = d2 < mg
                ig = jnp.where(upd, ics[u], ig)
                mg = jnp.minimum(d2, mg)
            new_ms[g] = mg
            new_is[g] = ig
        return tuple(new_ms), tuple(new_is)

    m0 = tuple(jnp.full((C_SUB, S_LANES), jnp.inf, jnp.float32)
               for _ in range(GROUPS))
    i0 = tuple(jnp.zeros((C_SUB, S_LANES), jnp.int32)
               for _ in range(GROUPS))
    ms, iss = lax.fori_loop(0, NS_PAD // (S_LANES * UNROLL), body, (m0, i0))

    big_i = jnp.int32(2 ** 30)
    for g in range(GROUPS):
        m = jnp.min(ms[g], axis=1, keepdims=True)
        isel = jnp.min(jnp.where(ms[g] == m, iss[g], big_i), axis=1,
                       keepdims=True)
        m_ref[0, 0, pl.ds(g * C_SUB, C_SUB)] = m
        idx_ref[0, 0, pl.ds(g * C_SUB, C_SUB)] = isel + bid * NS_PAD


def _nearest_v3(cloth, srep):
    B = cloth.shape[0]
    grid = (B, N_CTILES)
    out_shape = [
        jax.ShapeDtypeStruct((B, N_CTILES, C_PER_PROG, 1), jnp.float32),
        jax.ShapeDtypeStruct((B, N_CTILES, C_PER_PROG, 1), jnp.int32),
    ]
    m, idx = pl.pallas_call(
        _dist_kernel,
        grid=grid,
        in_specs=[
            pl.BlockSpec((1, C_PER_PROG, 3), lambda b, c: (b, c, 0)),
            pl.BlockSpec((1, 3 * C_SUB, NS_PAD), lambda b, c: (b, 0, 0)),
        ],
        out_specs=[
            pl.BlockSpec((1, 1, C_PER_PROG, 1), lambda b, c: (b, c, 0, 0)),
            pl.BlockSpec((1, 1, C_PER_PROG, 1), lambda b, c: (b, c, 0, 0)),
        ],
        out_shape=out_shape,
        scratch_shapes=[pltpu.VMEM((3 * C_PER_PROG, S_LANES), jnp.float32)],
        compiler_params=pltpu.CompilerParams(
            dimension_semantics=("parallel", "parallel"),
        ),
    )(cloth, srep)
    return m.reshape(B, -1), idx.reshape(B, -1)


N_IROWS = 8192 // 128


N_TILES_SC = 32
ROWS_PER_TILE = N_IROWS * 8 // N_TILES_SC
ELEMS_PER_TILE = ROWS_PER_TILE * 128


def _sc_loss_kernel(m_hbm, idx_hbm, sdf_hbm, lab_hbm, cvec_hbm, dt_hbm, st_hbm,
                    out_hbm, idx_v, gath_v, m_v, sdf_v, sc_v, sem):
    cid = lax.axis_index("c")
    sid = lax.axis_index("s")
    wid = cid * 16 + sid

    pltpu.sync_copy(idx_hbm.at[pl.ds(wid * ROWS_PER_TILE, ROWS_PER_TILE)],
                    idx_v)
    pltpu.sync_copy(m_hbm.at[pl.ds(wid * ELEMS_PER_TILE, ELEMS_PER_TILE)], m_v)
    pltpu.sync_copy(sdf_hbm.at[pl.ds(wid * ELEMS_PER_TILE, ELEMS_PER_TILE)],
                    sdf_v)
    pltpu.sync_copy(cvec_hbm, sc_v.at[0])
    pltpu.sync_copy(dt_hbm, sc_v.at[1])
    pltpu.sync_copy(st_hbm, sc_v.at[2])

    copies = [
        pltpu.async_copy(lab_hbm.at[idx_v.at[j]], gath_v.at[j], sem)
        for j in range(ROWS_PER_TILE)
    ]
    for c in copies:
        c.wait()

    cvec = sc_v[0]
    dt = sc_v[1]
    st = sc_v[2]
    dt2 = dt * dt

    def body(j, carry):
        acc, cnt = carry
        for k in range(8):
            lab = gath_v[j, pl.ds(k * 16, 16)]
            sl = pl.ds(j * 128 + k * 16, 16)
            mf = jnp.where(lab == cvec, 1.0, 0.0).astype(jnp.float32)
            d2 = m_v[sl]
            s = sdf_v[sl]
            nf = jnp.where(d2 < dt2, 1.0, 0.0).astype(jnp.float32)
            lp = jnp.abs(s) * mf
            ln = jnp.abs(s - st) * (1.0 - mf)
            acc = acc + (lp + ln) * nf
            cnt = cnt + mf
        return acc, cnt

    z = jnp.zeros((16,), jnp.float32)
    acc, cnt = lax.fori_loop(0, ROWS_PER_TILE, body, (z, z))
    sc_v[4] = acc
    sc_v[5] = cnt
    pltpu.sync_copy(sc_v.at[pl.ds(4, 2)], out_hbm.at[wid])


def _sc_loss(m, idx, sdf, lab, cvec, dtv, stv):
    B = sdf.shape[0]
    mesh = plsc.VectorSubcoreMesh(core_axis_name="c", subcore_axis_name="s")
    fn = functools.partial(
        pl.kernel,
        mesh=mesh,
        out_type=jax.ShapeDtypeStruct((N_TILES_SC, 2, 16), jnp.float32),
        scratch_types=[
            pltpu.VMEM((ROWS_PER_TILE, 128), jnp.int32),
            pltpu.VMEM((ROWS_PER_TILE, 128), jnp.float32),
            pltpu.VMEM((ELEMS_PER_TILE,), jnp.float32),
            pltpu.VMEM((ELEMS_PER_TILE,), jnp.float32),
            pltpu.VMEM((6, 16), jnp.float32),
            pltpu.SemaphoreType.DMA,
        ],
    )(_sc_loss_kernel)
    out = fn(m.reshape(-1), idx.reshape(B * N_IROWS, 128), sdf.reshape(-1),
             lab.reshape(-1), cvec, dtv, stv)
    out = out.reshape(B, N_TILES_SC // B, 2, 16)
    total = out[:, :, 0, :].sum(axis=(1, 2))
    n_in = out[:, :, 1, :].sum(axis=(1, 2))
    return total * (1.0 / 8192.0) * (n_in > 0.0).astype(jnp.float32)


def kernel(sdf, cloth_meshes_unposed, smpl_cloth_idx, smpl_cloth_valid,
           cloth_idx, sdf_thresh, dist_thresh, v_template):
    B, Nc, _ = cloth_meshes_unposed.shape
    Ns = v_template.shape[1]
    pad = NS_PAD - Ns

    FAR = jnp.float32(1e18)
    masked = jnp.where((smpl_cloth_valid > 0)[:, :, None], v_template, FAR)
    masked = jnp.pad(masked, ((0, 0), (0, pad), (0, 0)),
                     constant_values=1e18)
    smplt = jnp.swapaxes(masked, 1, 2)
    srep = jnp.broadcast_to(smplt[:, :, None, :],
                            (B, 3, C_SUB, NS_PAD)).reshape(B, 3 * C_SUB,
                                                           NS_PAD)

    m, idx = _nearest_v3(cloth_meshes_unposed, srep)

    lab = jnp.pad(smpl_cloth_idx, ((0, 0), (0, pad))).astype(jnp.float32)
    cvec = jnp.broadcast_to(cloth_idx[0].astype(jnp.float32), (16,))
    dtv = jnp.broadcast_to(dist_thresh.astype(jnp.float32), (16,))
    stv = jnp.broadcast_to(sdf_thresh.astype(jnp.float32), (16,))

    return _sc_loss(m, idx, sdf, lab, cvec, dtv, stv)
